# baseline (speedup 1.0000x reference)
import functools

import jax
import jax.numpy as jnp
from jax import lax
from jax.experimental import pallas as pl
from jax.experimental.pallas import tpu as pltpu
from jax.experimental.pallas import tpu_sc as plsc

N = 10000
E = 320000
G = 64
NSUB = 16
NCORE = 2
CH = 128
CHN = 158
CHNH = 79
EPAD = NSUB * CHN * CH
ACC_ROWS = 10112
RPT = ACC_ROWS // NSUB
LAST = N - (NSUB - 1) * RPT
NBLK = 10
BLK = N // NBLK


def _leaky(t):
    return jnp.where(t >= 0, t, 0.2 * t)


_MESH = plsc.VectorSubcoreMesh(core_axis_name="c", subcore_axis_name="s")
_SC_PARAMS = pltpu.CompilerParams(use_tc_tiling_on_sc=False)


def _zero_acc(acc, zeros, row0):
    pltpu.sync_copy(zeros.at[pl.ds(row0, RPT)], acc.at[pl.ds(row0, RPT)])


def _copy_out(acc, outq, sid, row0):
    @pl.when(sid < NSUB - 1)
    def _():
        pltpu.sync_copy(acc.at[pl.ds(row0, RPT)], outq.at[pl.ds(row0, RPT)])

    @pl.when(sid == NSUB - 1)
    def _():
        pltpu.sync_copy(acc.at[pl.ds(row0, LAST)], outq.at[pl.ds(row0, LAST)])


@functools.partial(
    pl.kernel, mesh=_MESH, compiler_params=_SC_PARAMS,
    out_type=jax.ShapeDtypeStruct((4, N, 64), jnp.float32),
    scratch_types=[
        pltpu.VMEM((CHN, CH), jnp.int32),
        pltpu.VMEM((CHN, CH), jnp.int32),
        pltpu.VMEM((CH, 64), jnp.float32),
        pltpu.VMEM_SHARED((ACC_ROWS, 64), jnp.float32),
        pltpu.SemaphoreType.DMA,
    ],
)
def _seg_h(tbl, srci, dsti, zeros, out, srcv, dstv, rows, acc, sem):
    cid = lax.axis_index("c")
    sid = lax.axis_index("s")
    row0 = sid * RPT
    pltpu.sync_copy(srci.at[sid], srcv)
    pltpu.sync_copy(dsti.at[sid], dstv)
    for phase in range(2):
        q = phase * NCORE + cid
        _zero_acc(acc, zeros, row0)
        plsc.subcore_barrier()
        tblq = tbl.at[q]

        def body(j, carry):
            pltpu.async_copy(tblq.at[srcv.at[j]], rows, sem).wait()
            pltpu.sync_copy(rows, acc.at[dstv.at[j]], add=True)
            return carry

        lax.fori_loop(0, CHN, body, 0)
        plsc.subcore_barrier()
        _copy_out(acc, out.at[q], sid, row0)
        if phase == 0:
            plsc.subcore_barrier()


@functools.partial(
    pl.kernel, mesh=_MESH, compiler_params=_SC_PARAMS,
    out_type=jax.ShapeDtypeStruct((NCORE, N, 64), jnp.float32),
    scratch_types=[
        pltpu.VMEM((CHNH, CH), jnp.int32),
        pltpu.VMEM((CHNH, CH), jnp.int32),
        pltpu.VMEM((CH, 64), jnp.float32),
        pltpu.VMEM_SHARED((ACC_ROWS, 64), jnp.float32),
        pltpu.SemaphoreType.DMA,
    ],
)
def _seg_lat(tbl, srci, dsti, zeros, out, srcv, dstv, rows, acc, sem):
    cid = lax.axis_index("c")
    sid = lax.axis_index("s")
    row0 = sid * RPT
    _zero_acc(acc, zeros, row0)
    pltpu.sync_copy(srci.at[cid, sid], srcv)
    pltpu.sync_copy(dsti.at[cid, sid], dstv)
    plsc.subcore_barrier()

    def body(j, carry):
        pltpu.async_copy(tbl.at[srcv.at[j]], rows, sem).wait()
        pltpu.sync_copy(rows, acc.at[dstv.at[j]], add=True)
        return carry

    lax.fori_loop(0, CHNH, body, 0)
    plsc.subcore_barrier()
    _copy_out(acc, out.at[cid], sid, row0)


@functools.partial(
    pl.kernel, mesh=_MESH, compiler_params=_SC_PARAMS,
    out_type=jax.ShapeDtypeStruct((NCORE, N, 32), jnp.float32),
    scratch_types=[
        pltpu.VMEM((CHNH, CH), jnp.int32),
        pltpu.VMEM((CH, 32), jnp.float32),
        pltpu.VMEM_SHARED((ACC_ROWS, 32), jnp.float32),
        pltpu.SemaphoreType.DMA,
    ],
)
def _seg_e(tbl, dsti, zeros, out, dstv, rows, acc, sem):
    cid = lax.axis_index("c")
    sid = lax.axis_index("s")
    row0 = sid * RPT
    _zero_acc(acc, zeros, row0)
    pltpu.sync_copy(dsti.at[cid, sid], dstv)
    plsc.subcore_barrier()

    def body(j, carry):
        base = ((cid * NSUB + sid) * CHNH + j) * CH
        pltpu.sync_copy(tbl.at[pl.ds(base, CH)], rows)
        pltpu.sync_copy(rows, acc.at[dstv.at[j]], add=True)
        return carry

    lax.fori_loop(0, CHNH, body, 0)
    plsc.subcore_barrier()
    _copy_out(acc, out.at[cid], sid, row0)


def _cat4(ref):
    return jnp.concatenate([ref[0], ref[1], ref[2], ref[3]], axis=-1)


def _split4(o_ref, t):
    for qq in range(4):
        o_ref[qq] = t[:, qq * 64:(qq + 1) * 64]


def _tc_h0(x, Wn, bn):
    def body(x_ref, w_ref, b_ref, o_ref):
        t = jnp.dot(x_ref[...], w_ref[...], preferred_element_type=jnp.float32)
        _split4(o_ref, t + b_ref[...][None, :])

    return pl.pallas_call(
        body,
        grid=(NBLK,),
        in_specs=[
            pl.BlockSpec((BLK, 128), lambda i: (i, 0)),
            pl.BlockSpec((128, 256), lambda i: (0, 0)),
            pl.BlockSpec((256,), lambda i: (0,)),
        ],
        out_specs=pl.BlockSpec((4, BLK, 64), lambda i: (0, i, 0)),
        out_shape=jax.ShapeDtypeStruct((4, N, 64), jnp.float32),
    )(x, Wn, bn)


def _tc_prep(eraw, We, be, batchc, latents):
    def body(er_ref, we_ref, be_ref, b_ref, lat_ref, eagg_ref, latn_ref):
        er = er_ref[0] + er_ref[1]
        ea = er[:, 0:16]
        deg = er[:, 16:17]
        eagg = jnp.dot(ea, we_ref[...], preferred_element_type=jnp.float32)
        eagg_ref[...] = eagg + deg * be_ref[...][None, :]
        bcol = b_ref[:, 0:1]
        oh = (bcol == lax.broadcasted_iota(jnp.int32, (BLK, G), 1))
        latn_ref[...] = jnp.dot(oh.astype(jnp.float32), lat_ref[...],
                                preferred_element_type=jnp.float32)

    return pl.pallas_call(
        body,
        grid=(NBLK,),
        in_specs=[
            pl.BlockSpec((2, BLK, 32), lambda i: (0, i, 0)),
            pl.BlockSpec((16, 64), lambda i: (0, 0)),
            pl.BlockSpec((64,), lambda i: (0,)),
            pl.BlockSpec((BLK, 8), lambda i: (i, 0)),
            pl.BlockSpec((G, 64), lambda i: (0, 0)),
        ],
        out_specs=[
            pl.BlockSpec((BLK, 64), lambda i: (i, 0)),
            pl.BlockSpec((BLK, 64), lambda i: (i, 0)),
        ],
        out_shape=[
            jax.ShapeDtypeStruct((N, 64), jnp.float32),
            jax.ShapeDtypeStruct((N, 64), jnp.float32),
        ],
    )(eraw, We, be, batchc, latents)


def _tc_bb(ah, h, eagg, Wm, Wu):
    def body(ah_ref, h_ref, e_ref, wm_ref, wu_ref, o_ref):
        ahc = _cat4(ah_ref)
        agg = jnp.dot(ahc, wm_ref[0:256], preferred_element_type=jnp.float32)
        agg += jnp.dot(e_ref[...], wm_ref[256:320], preferred_element_type=jnp.float32)
        t = jnp.dot(agg, wu_ref[...], preferred_element_type=jnp.float32)
        _split4(o_ref, _leaky(t + _cat4(h_ref)))

    return pl.pallas_call(
        body,
        grid=(NBLK,),
        in_specs=[
            pl.BlockSpec((4, BLK, 64), lambda i: (0, i, 0)),
            pl.BlockSpec((4, BLK, 64), lambda i: (0, i, 0)),
            pl.BlockSpec((BLK, 64), lambda i: (i, 0)),
            pl.BlockSpec((320, 256), lambda i: (0, 0)),
            pl.BlockSpec((256, 256), lambda i: (0, 0)),
        ],
        out_specs=pl.BlockSpec((4, BLK, 64), lambda i: (0, i, 0)),
        out_shape=jax.ShapeDtypeStruct((4, N, 64), jnp.float32),
    )(ah, h, eagg, Wm, Wu)


def _tc_disc(ah, lagg, eagg, Wdi):
    def body(ah_ref, l_ref, e_ref, wd_ref, o_ref):
        ahc = _cat4(ah_ref)
        agg = jnp.dot(ahc, wd_ref[0:256], preferred_element_type=jnp.float32)
        lsum = l_ref[0] + l_ref[1]
        agg += jnp.dot(lsum, wd_ref[256:320], preferred_element_type=jnp.float32)
        agg += jnp.dot(e_ref[...], wd_ref[320:384], preferred_element_type=jnp.float32)
        _split4(o_ref, _leaky(agg))

    return pl.pallas_call(
        body,
        grid=(NBLK,),
        in_specs=[
            pl.BlockSpec((4, BLK, 64), lambda i: (0, i, 0)),
            pl.BlockSpec((2, BLK, 64), lambda i: (0, i, 0)),
            pl.BlockSpec((BLK, 64), lambda i: (i, 0)),
            pl.BlockSpec((384, 256), lambda i: (0, 0)),
        ],
        out_specs=pl.BlockSpec((4, BLK, 64), lambda i: (0, i, 0)),
        out_shape=jax.ShapeDtypeStruct((4, N, 64), jnp.float32),
    )(ah, lagg, eagg, Wdi)


def _tc_pool(h, batch3, batchc, W1, b1, W2p, b2p):
    def body(h_ref, b_ref, bc_ref, w1_ref, b1_ref, w2_ref, b2_ref, o_ref,
             mx, sm, cnt):
        i = pl.program_id(0)

        @pl.when(i == 0)
        def _():
            mx[...] = jnp.full((G, 256), -jnp.inf, jnp.float32)
            sm[...] = jnp.zeros((G, 256), jnp.float32)
            cnt[...] = jnp.zeros((G, 128), jnp.float32)

        hb = _cat4(h_ref)
        bbr = b_ref[0]
        oht = (bbr == lax.broadcasted_iota(jnp.int32, (G, BLK), 0))
        ohtf = oht.astype(jnp.float32)
        sm[...] += jnp.dot(ohtf, hb, preferred_element_type=jnp.float32)
        cnt[...] += jnp.broadcast_to(
            jnp.sum(ohtf, axis=1, keepdims=True), (G, 128))

        bcol = bc_ref[:, 0:1]
        neg = jnp.float32(-jnp.inf)

        def gbody(gi, carry):
            mask = bcol == gi
            m = jnp.max(jnp.where(mask, hb, neg), axis=0, keepdims=True)
            mx[pl.ds(gi, 1), :] = jnp.maximum(mx[pl.ds(gi, 1), :], m)
            return carry

        lax.fori_loop(0, G, gbody, 0)

        @pl.when(i == NBLK - 1)
        def _():
            c = jnp.maximum(cnt[...], 1.0)
            mean = sm[...] / jnp.concatenate([c, c], axis=-1)
            t = jnp.dot(mx[...], w1_ref[0:256], preferred_element_type=jnp.float32)
            t += jnp.dot(mean, w1_ref[256:512], preferred_element_type=jnp.float32)
            t = _leaky(t + b1_ref[...][None, :])
            y = jnp.dot(t, w2_ref[...], preferred_element_type=jnp.float32)
            o_ref[...] = y + b2_ref[...][None, :]

    return pl.pallas_call(
        body,
        grid=(NBLK,),
        in_specs=[
            pl.BlockSpec((4, BLK, 64), lambda i: (0, i, 0)),
            pl.BlockSpec((1, 1, BLK), lambda i: (i, 0, 0)),
            pl.BlockSpec((BLK, 8), lambda i: (i, 0)),
            pl.BlockSpec((512, 256), lambda i: (0, 0)),
            pl.BlockSpec((256,), lambda i: (0,)),
            pl.BlockSpec((256, 128), lambda i: (0, 0)),
            pl.BlockSpec((128,), lambda i: (0,)),
        ],
        out_specs=pl.BlockSpec((G, 128), lambda i: (0, 0)),
        out_shape=jax.ShapeDtypeStruct((G, 128), jnp.float32),
        scratch_shapes=[
            pltpu.VMEM((G, 256), jnp.float32),
            pltpu.VMEM((G, 256), jnp.float32),
            pltpu.VMEM((G, 128), jnp.float32),
        ],
    )(h, batch3, batchc, W1, b1, W2p, b2p)


def kernel(x, edge_index, edge_attr, batch, latents,
           Wn, bn, We, be, Wmsg, Wupd, Wd, W1, b1, W2, b2):
    src = edge_index[0].astype(jnp.int32)
    dst = edge_index[1].astype(jnp.int32)
    padn = EPAD - E
    srcp = jnp.concatenate([src, jnp.zeros((padn,), jnp.int32)])
    dstp = jnp.concatenate([dst, jnp.full((padn,), N, jnp.int32)])
    srch = srcp.reshape(NSUB, CHN, CH)
    dsth = dstp.reshape(NSUB, CHN, CH)
    srcq = srcp.reshape(NCORE, NSUB, CHNH, CH)
    dstq = dstp.reshape(NCORE, NSUB, CHNH, CH)

    z64 = jnp.zeros((ACC_ROWS, 64), jnp.float32)
    z32 = jnp.zeros((ACC_ROWS, 32), jnp.float32)

    ea_pad = jnp.concatenate([edge_attr, jnp.zeros((padn, 16), jnp.float32)])
    ones_col = jnp.concatenate(
        [jnp.ones((E, 1), jnp.float32), jnp.zeros((padn, 1), jnp.float32)])
    etbl = jnp.concatenate(
        [ea_pad, ones_col, jnp.zeros((EPAD, 15), jnp.float32)], axis=1)

    batch32 = batch.astype(jnp.int32)
    batch3 = batch32.reshape(NBLK, 1, BLK)
    batchc = jnp.broadcast_to(batch32[:, None], (N, 8))

    eraw = _seg_e(etbl, dstq, z32)
    h = _tc_h0(x, Wn, bn)
    eagg, latn = _tc_prep(eraw, We, be, batchc, latents)
    lagg = _seg_lat(latn, srcq, dstq, z64)

    for i in range(6):
        ah = _seg_h(h, srch, dsth, z64)
        h = _tc_bb(ah, h, eagg, Wmsg[i], Wupd[i])
    for i in range(3):
        ah = _seg_h(h, srch, dsth, z64)
        h = _tc_disc(ah, lagg, eagg, Wd[i])

    W2p = jnp.concatenate([W2, jnp.zeros((256, 127), jnp.float32)], axis=1)
    b2p = jnp.concatenate([b2, jnp.zeros((127,), jnp.float32)])
    y = _tc_pool(h, batch3, batchc, W1, b1, W2p, b2p)
    return y[:, 0]


# speedup vs baseline: 1.1549x; 1.1549x over previous
import functools

import jax
import jax.numpy as jnp
from jax import lax
from jax.experimental import pallas as pl
from jax.experimental.pallas import tpu as pltpu
from jax.experimental.pallas import tpu_sc as plsc

N = 10000
E = 320000
G = 64
NSUB = 16
NCORE = 2
CH = 128
CHUNKS = 157
EPAD = NSUB * CHUNKS * CH
ACC_ROWS = 10112
RPT = ACC_ROWS // NSUB
LAST = N - (NSUB - 1) * RPT
NBLK = 10
BLK = N // NBLK


def _leaky(t):
    return jnp.where(t >= 0, t, 0.2 * t)


_MESH = plsc.VectorSubcoreMesh(core_axis_name="c", subcore_axis_name="s")
_SC_PARAMS = pltpu.CompilerParams(use_tc_tiling_on_sc=False)


def _zero_loop_copy(acc, zeros, sid, row0):
    pltpu.sync_copy(zeros.at[pl.ds(row0, RPT)], acc.at[pl.ds(row0, RPT)])


def _copy_out(acc, outq, sid, row0):
    @pl.when(sid < NSUB - 1)
    def _():
        pltpu.sync_copy(acc.at[pl.ds(row0, RPT)], outq.at[pl.ds(row0, RPT)])

    @pl.when(sid == NSUB - 1)
    def _():
        pltpu.sync_copy(acc.at[pl.ds(row0, LAST)], outq.at[pl.ds(row0, LAST)])


@functools.partial(
    pl.kernel, mesh=_MESH, compiler_params=_SC_PARAMS,
    out_type=jax.ShapeDtypeStruct((4, N, 64), jnp.float32),
    scratch_types=[
        pltpu.VMEM((CHUNKS, CH), jnp.int32),
        pltpu.VMEM((CHUNKS, CH), jnp.int32),
        pltpu.VMEM((CH, 64), jnp.float32),
        pltpu.VMEM_SHARED((ACC_ROWS, 64), jnp.float32),
        pltpu.SemaphoreType.DMA,
    ],
)
def _seg_h(tbl, srci, dsti, zeros, out, srcv, dstv, rows, acc, sem):
    cid = lax.axis_index("c")
    sid = lax.axis_index("s")
    row0 = sid * RPT
    pltpu.sync_copy(srci.at[sid], srcv)
    pltpu.sync_copy(dsti.at[sid], dstv)
    for half in range(2):
        q = half * 2 + cid
        _zero_loop_copy(acc, zeros, sid, row0)
        plsc.subcore_barrier()
        tblq = tbl.at[q]

        def body(j, carry):
            pltpu.async_copy(tblq.at[srcv.at[j]], rows, sem).wait()
            pltpu.sync_copy(rows, acc.at[dstv.at[j]], add=True)
            return carry

        lax.fori_loop(0, CHUNKS, body, 0)
        plsc.subcore_barrier()
        _copy_out(acc, out.at[q], sid, row0)
        plsc.subcore_barrier()


@functools.partial(
    pl.kernel, mesh=_MESH, compiler_params=_SC_PARAMS,
    out_type=jax.ShapeDtypeStruct((NCORE, N, 32), jnp.float32),
    scratch_types=[
        pltpu.VMEM((CHUNKS, CH), jnp.int32),
        pltpu.VMEM((CHUNKS, CH), jnp.int32),
        pltpu.VMEM((CH, 32), jnp.float32),
        pltpu.VMEM_SHARED((ACC_ROWS, 32), jnp.float32),
        pltpu.SemaphoreType.DMA,
    ],
)
def _seg_lat(tbl, srci, dsti, zeros, out, srcv, dstv, rows, acc, sem):
    cid = lax.axis_index("c")
    sid = lax.axis_index("s")
    row0 = sid * RPT
    _zero_loop_copy(acc, zeros, sid, row0)
    pltpu.sync_copy(srci.at[sid], srcv)
    pltpu.sync_copy(dsti.at[sid], dstv)
    plsc.subcore_barrier()
    tblc = tbl.at[cid]

    def body(j, carry):
        pltpu.async_copy(tblc.at[srcv.at[j]], rows, sem).wait()
        pltpu.sync_copy(rows, acc.at[dstv.at[j]], add=True)
        return carry

    lax.fori_loop(0, CHUNKS, body, 0)
    plsc.subcore_barrier()
    _copy_out(acc, out.at[cid], sid, row0)


@functools.partial(
    pl.kernel, mesh=_MESH, compiler_params=_SC_PARAMS,
    out_type=jax.ShapeDtypeStruct((NCORE, N, 16), jnp.float32),
    scratch_types=[
        pltpu.VMEM((CHUNKS, CH), jnp.int32),
        pltpu.VMEM((CH, 16), jnp.float32),
        pltpu.VMEM_SHARED((ACC_ROWS, 16), jnp.float32),
        pltpu.SemaphoreType.DMA,
    ],
)
def _seg_e(tbl, dsti, zeros, out, dstv, rows, acc, sem):
    cid = lax.axis_index("c")
    sid = lax.axis_index("s")
    row0 = sid * RPT
    _zero_loop_copy(acc, zeros, sid, row0)
    pltpu.sync_copy(dsti.at[sid], dstv)
    plsc.subcore_barrier()
    tblc = tbl.at[cid]

    def body(j, carry):
        base = (sid * CHUNKS + j) * CH
        pltpu.sync_copy(tblc.at[pl.ds(base, CH)], rows)
        pltpu.sync_copy(rows, acc.at[dstv.at[j]], add=True)
        return carry

    lax.fori_loop(0, CHUNKS, body, 0)
    plsc.subcore_barrier()
    _copy_out(acc, out.at[cid], sid, row0)


def _cat4(ref):
    return jnp.concatenate([ref[0], ref[1], ref[2], ref[3]], axis=-1)


def _split4(o_ref, t):
    for q in range(4):
        o_ref[q] = t[:, q * 64:(q + 1) * 64]


def _tc_h0(x, Wn, bn):
    def body(x_ref, w_ref, b_ref, o_ref):
        t = jnp.dot(x_ref[...], w_ref[...], preferred_element_type=jnp.float32)
        _split4(o_ref, t + b_ref[...][None, :])

    return pl.pallas_call(
        body,
        grid=(NBLK,),
        in_specs=[
            pl.BlockSpec((BLK, 128), lambda i: (i, 0)),
            pl.BlockSpec((128, 256), lambda i: (0, 0)),
            pl.BlockSpec((256,), lambda i: (0,)),
        ],
        out_specs=pl.BlockSpec((4, BLK, 64), lambda i: (0, i, 0)),
        out_shape=jax.ShapeDtypeStruct((4, N, 64), jnp.float32),
    )(x, Wn, bn)


def _tc_prep(eraw, We, be, batchc, latents):
    def body(er_ref, we_ref, be_ref, b_ref, lat_ref, eagg_ref, latn_ref):
        ea = er_ref[0]
        deg = er_ref[1][:, 0:1]
        eagg = jnp.dot(ea, we_ref[...], preferred_element_type=jnp.float32)
        eagg_ref[...] = eagg + deg * be_ref[...][None, :]
        bcol = b_ref[:, 0:1]
        oh = (bcol == lax.broadcasted_iota(jnp.int32, (BLK, G), 1))
        latn = jnp.dot(oh.astype(jnp.float32), lat_ref[...],
                       preferred_element_type=jnp.float32)
        latn_ref[0] = latn[:, :32]
        latn_ref[1] = latn[:, 32:]

    return pl.pallas_call(
        body,
        grid=(NBLK,),
        in_specs=[
            pl.BlockSpec((2, BLK, 16), lambda i: (0, i, 0)),
            pl.BlockSpec((16, 64), lambda i: (0, 0)),
            pl.BlockSpec((64,), lambda i: (0,)),
            pl.BlockSpec((BLK, 8), lambda i: (i, 0)),
            pl.BlockSpec((G, 64), lambda i: (0, 0)),
        ],
        out_specs=[
            pl.BlockSpec((BLK, 64), lambda i: (i, 0)),
            pl.BlockSpec((2, BLK, 32), lambda i: (0, i, 0)),
        ],
        out_shape=[
            jax.ShapeDtypeStruct((N, 64), jnp.float32),
            jax.ShapeDtypeStruct((2, N, 32), jnp.float32),
        ],
    )(eraw, We, be, batchc, latents)


def _tc_bb(ah, h, eagg, Wm, Wu):
    def body(ah_ref, h_ref, e_ref, wm_ref, wu_ref, o_ref):
        ahc = _cat4(ah_ref)
        agg = jnp.dot(ahc, wm_ref[0:256], preferred_element_type=jnp.float32)
        agg += jnp.dot(e_ref[...], wm_ref[256:320], preferred_element_type=jnp.float32)
        t = jnp.dot(agg, wu_ref[...], preferred_element_type=jnp.float32)
        _split4(o_ref, _leaky(t + _cat4(h_ref)))

    return pl.pallas_call(
        body,
        grid=(NBLK,),
        in_specs=[
            pl.BlockSpec((4, BLK, 64), lambda i: (0, i, 0)),
            pl.BlockSpec((4, BLK, 64), lambda i: (0, i, 0)),
            pl.BlockSpec((BLK, 64), lambda i: (i, 0)),
            pl.BlockSpec((320, 256), lambda i: (0, 0)),
            pl.BlockSpec((256, 256), lambda i: (0, 0)),
        ],
        out_specs=pl.BlockSpec((4, BLK, 64), lambda i: (0, i, 0)),
        out_shape=jax.ShapeDtypeStruct((4, N, 64), jnp.float32),
    )(ah, h, eagg, Wm, Wu)


def _tc_disc(ah, lagg, eagg, Wdi):
    def body(ah_ref, l_ref, e_ref, wd_ref, o_ref):
        ahc = _cat4(ah_ref)
        agg = jnp.dot(ahc, wd_ref[0:256], preferred_element_type=jnp.float32)
        lc = jnp.concatenate([l_ref[0], l_ref[1]], axis=-1)
        agg += jnp.dot(lc, wd_ref[256:320], preferred_element_type=jnp.float32)
        agg += jnp.dot(e_ref[...], wd_ref[320:384], preferred_element_type=jnp.float32)
        _split4(o_ref, _leaky(agg))

    return pl.pallas_call(
        body,
        grid=(NBLK,),
        in_specs=[
            pl.BlockSpec((4, BLK, 64), lambda i: (0, i, 0)),
            pl.BlockSpec((2, BLK, 32), lambda i: (0, i, 0)),
            pl.BlockSpec((BLK, 64), lambda i: (i, 0)),
            pl.BlockSpec((384, 256), lambda i: (0, 0)),
        ],
        out_specs=pl.BlockSpec((4, BLK, 64), lambda i: (0, i, 0)),
        out_shape=jax.ShapeDtypeStruct((4, N, 64), jnp.float32),
    )(ah, lagg, eagg, Wdi)


def _tc_pool(h, batch3, batchc, W1, b1, W2p, b2p):
    def body(h_ref, b_ref, bc_ref, w1_ref, b1_ref, w2_ref, b2_ref, o_ref,
             mx, sm, cnt):
        i = pl.program_id(0)

        @pl.when(i == 0)
        def _():
            mx[...] = jnp.full((G, 256), -jnp.inf, jnp.float32)
            sm[...] = jnp.zeros((G, 256), jnp.float32)
            cnt[...] = jnp.zeros((G, 128), jnp.float32)

        hb = _cat4(h_ref)
        bbr = b_ref[0]
        oht = (bbr == lax.broadcasted_iota(jnp.int32, (G, BLK), 0))
        ohtf = oht.astype(jnp.float32)
        sm[...] += jnp.dot(ohtf, hb, preferred_element_type=jnp.float32)
        cnt[...] += jnp.broadcast_to(
            jnp.sum(ohtf, axis=1, keepdims=True), (G, 128))

        bcol = bc_ref[:, 0:1]
        neg = jnp.float32(-jnp.inf)

        def gbody(gi, carry):
            mask = bcol == gi
            m = jnp.max(jnp.where(mask, hb, neg), axis=0, keepdims=True)
            mx[pl.ds(gi, 1), :] = jnp.maximum(mx[pl.ds(gi, 1), :], m)
            return carry

        lax.fori_loop(0, G, gbody, 0)

        @pl.when(i == NBLK - 1)
        def _():
            c = jnp.maximum(cnt[...], 1.0)
            mean = sm[...] / jnp.concatenate([c, c], axis=-1)
            t = jnp.dot(mx[...], w1_ref[0:256], preferred_element_type=jnp.float32)
            t += jnp.dot(mean, w1_ref[256:512], preferred_element_type=jnp.float32)
            t = _leaky(t + b1_ref[...][None, :])
            y = jnp.dot(t, w2_ref[...], preferred_element_type=jnp.float32)
            o_ref[...] = y + b2_ref[...][None, :]

    return pl.pallas_call(
        body,
        grid=(NBLK,),
        in_specs=[
            pl.BlockSpec((4, BLK, 64), lambda i: (0, i, 0)),
            pl.BlockSpec((1, 1, BLK), lambda i: (i, 0, 0)),
            pl.BlockSpec((BLK, 8), lambda i: (i, 0)),
            pl.BlockSpec((512, 256), lambda i: (0, 0)),
            pl.BlockSpec((256,), lambda i: (0,)),
            pl.BlockSpec((256, 128), lambda i: (0, 0)),
            pl.BlockSpec((128,), lambda i: (0,)),
        ],
        out_specs=pl.BlockSpec((G, 128), lambda i: (0, 0)),
        out_shape=jax.ShapeDtypeStruct((G, 128), jnp.float32),
        scratch_shapes=[
            pltpu.VMEM((G, 256), jnp.float32),
            pltpu.VMEM((G, 256), jnp.float32),
            pltpu.VMEM((G, 128), jnp.float32),
        ],
    )(h, batch3, batchc, W1, b1, W2p, b2p)


def kernel(x, edge_index, edge_attr, batch, latents,
           Wn, bn, We, be, Wmsg, Wupd, Wd, W1, b1, W2, b2):
    src = edge_index[0].astype(jnp.int32)
    dst = edge_index[1].astype(jnp.int32)
    padn = EPAD - E
    srcp = jnp.concatenate([src, jnp.zeros((padn,), jnp.int32)])
    srcp = srcp.reshape(NSUB, CHUNKS, CH)
    dstp = jnp.concatenate([dst, jnp.full((padn,), N, jnp.int32)])
    dstp = dstp.reshape(NSUB, CHUNKS, CH)

    z64 = jnp.zeros((ACC_ROWS, 64), jnp.float32)
    z32 = jnp.zeros((ACC_ROWS, 32), jnp.float32)
    z16 = jnp.zeros((ACC_ROWS, 16), jnp.float32)

    ea_pad = jnp.concatenate([edge_attr, jnp.zeros((padn, 16), jnp.float32)])
    ones_col = jnp.concatenate(
        [jnp.ones((E, 1), jnp.float32), jnp.zeros((padn, 1), jnp.float32)])
    etbl = jnp.stack(
        [ea_pad, jnp.concatenate([ones_col, jnp.zeros((EPAD, 15), jnp.float32)],
                                 axis=1)], axis=0)

    batch32 = batch.astype(jnp.int32)
    batch3 = batch32.reshape(NBLK, 1, BLK)
    batchc = jnp.broadcast_to(batch32[:, None], (N, 8))

    eraw = _seg_e(etbl, dstp, z16)
    h = _tc_h0(x, Wn, bn)
    eagg, latn = _tc_prep(eraw, We, be, batchc, latents)
    lagg = _seg_lat(latn, srcp, dstp, z32)

    for i in range(6):
        ah = _seg_h(h, srcp, dstp, z64)
        h = _tc_bb(ah, h, eagg, Wmsg[i], Wupd[i])
    for i in range(3):
        ah = _seg_h(h, srcp, dstp, z64)
        h = _tc_disc(ah, lagg, eagg, Wd[i])

    W2p = jnp.concatenate([W2, jnp.zeros((256, 127), jnp.float32)], axis=1)
    b2p = jnp.concatenate([b2, jnp.zeros((127,), jnp.float32)])
    y = _tc_pool(h, batch3, batchc, W1, b1, W2p, b2p)
    return y[:, 0]
